# baseline (speedup 1.0000x reference)
import functools

import jax
import jax.numpy as jnp
from jax.experimental import pallas as pl


def _pre_body(s_ref, a_ref, ws_ref, wa_ref, atts_ref, attd_ref,
              h_ref, asrc_ref, adst_ref):
    hb = (jnp.dot(s_ref[...], ws_ref[...], preferred_element_type=jnp.float32)
          + jnp.dot(a_ref[...], wa_ref[...], preferred_element_type=jnp.float32))
    h_ref[...] = hb
    asrc_ref[...] = jnp.sum(hb * atts_ref[...], axis=1, keepdims=True)
    adst_ref[...] = jnp.sum(hb * attd_ref[...], axis=1, keepdims=True)


def _post_body(g_ref, w1_ref, b1_ref, w2_ref, b2_ref, w3_ref, b3_ref,
               v1_ref, c1_ref, v2_ref, c2_ref, v3_ref, c3_ref,
               q1_ref, q2_ref):
    g = g_ref[...]
    h1 = jax.nn.relu(jnp.dot(g, w1_ref[...], preferred_element_type=jnp.float32)
                     + b1_ref[...])
    h1 = jax.nn.relu(jnp.dot(h1, w2_ref[...], preferred_element_type=jnp.float32)
                     + b2_ref[...])
    q1_ref[...] = jnp.dot(h1, w3_ref[...], preferred_element_type=jnp.float32) + b3_ref[...]
    h2 = jax.nn.relu(jnp.dot(g, v1_ref[...], preferred_element_type=jnp.float32)
                     + c1_ref[...])
    h2 = jax.nn.relu(jnp.dot(h2, v2_ref[...], preferred_element_type=jnp.float32)
                     + c2_ref[...])
    q2_ref[...] = jnp.dot(h2, v3_ref[...], preferred_element_type=jnp.float32) + c3_ref[...]


def kernel(s, a, edge_index, W_gat, att_src, att_dst, b_gat,
           W1, b1, W2, b2, W3, b3, V1, c1, V2, c2, V3, c3):
    n, obs = s.shape
    act = a.shape[1]
    hdim = W_gat.shape[1]
    Ws = W_gat[:obs]
    Wa = W_gat[obs:]
    B = 2000
    grid = (n // B,)

    full = lambda shp: pl.BlockSpec(shp, lambda i: tuple(0 for _ in shp))
    h, asrc, adst = pl.pallas_call(
        _pre_body,
        grid=grid,
        in_specs=[
            pl.BlockSpec((B, obs), lambda i: (i, 0)),
            pl.BlockSpec((B, act), lambda i: (i, 0)),
            full((obs, hdim)), full((act, hdim)),
            full((1, hdim)), full((1, hdim)),
        ],
        out_specs=[
            pl.BlockSpec((B, hdim), lambda i: (i, 0)),
            pl.BlockSpec((B, 1), lambda i: (i, 0)),
            pl.BlockSpec((B, 1), lambda i: (i, 0)),
        ],
        out_shape=[
            jax.ShapeDtypeStruct((n, hdim), jnp.float32),
            jax.ShapeDtypeStruct((n, 1), jnp.float32),
            jax.ShapeDtypeStruct((n, 1), jnp.float32),
        ],
    )(s, a, Ws, Wa, att_src.reshape(1, hdim), att_dst.reshape(1, hdim))
    asrc = asrc.reshape(n)
    adst = adst.reshape(n)

    loops = jnp.arange(n, dtype=edge_index.dtype)
    src = jnp.concatenate([edge_index[0], loops])
    dst = jnp.concatenate([edge_index[1], loops])
    e = asrc[src] + adst[dst]
    e = jnp.where(e > 0, e, 0.2 * e)
    emax = jax.ops.segment_max(e, dst, num_segments=n)
    emax = jnp.where(jnp.isfinite(emax), emax, 0.0)
    w = jnp.exp(e - emax[dst])
    denom = jax.ops.segment_sum(w, dst, num_segments=n)
    alpha = w / (denom[dst] + 1e-16)
    msg = h[src] * alpha[:, None]
    g = jax.ops.segment_sum(msg, dst, num_segments=n) + b_gat

    q1, q2 = pl.pallas_call(
        _post_body,
        grid=grid,
        in_specs=[
            pl.BlockSpec((B, hdim), lambda i: (i, 0)),
            full((hdim, hdim)), full((1, hdim)),
            full((hdim, hdim)), full((1, hdim)),
            full((hdim, 1)), full((1, 1)),
            full((hdim, hdim)), full((1, hdim)),
            full((hdim, hdim)), full((1, hdim)),
            full((hdim, 1)), full((1, 1)),
        ],
        out_specs=[
            pl.BlockSpec((B, 1), lambda i: (i, 0)),
            pl.BlockSpec((B, 1), lambda i: (i, 0)),
        ],
        out_shape=[
            jax.ShapeDtypeStruct((n, 1), jnp.float32),
            jax.ShapeDtypeStruct((n, 1), jnp.float32),
        ],
    )(g, W1, b1.reshape(1, hdim), W2, b2.reshape(1, hdim), W3, b3.reshape(1, 1),
      V1, c1.reshape(1, hdim), V2, c2.reshape(1, hdim), V3, c3.reshape(1, 1))
    return (q1, q2)


# speedup vs baseline: 19.9356x; 19.9356x over previous
import functools

import jax
import jax.numpy as jnp
from jax import lax
from jax.experimental import pallas as pl
from jax.experimental.pallas import tpu as pltpu
from jax.experimental.pallas import tpu_sc as plsc

_WIN = 128
_NTILE = 16
_NCORE = 2
_SC_PARAMS = pltpu.CompilerParams(use_tc_tiling_on_sc=False,
                                  needs_layout_passes=False)


def _pre_body(s_ref, a_ref, ws_ref, wa_ref, atts_ref, attd_ref,
              hlo_ref, hhi_ref, asrc_ref, adst_ref):
    hb = (jnp.dot(s_ref[...], ws_ref[...], preferred_element_type=jnp.float32)
          + jnp.dot(a_ref[...], wa_ref[...], preferred_element_type=jnp.float32))
    hlo_ref[...] = hb[:, :16]
    hhi_ref[...] = hb[:, 16:]
    asrc_ref[...] = jnp.sum(hb * atts_ref[...], axis=1, keepdims=True)
    adst_ref[...] = jnp.sum(hb * attd_ref[...], axis=1, keepdims=True)


def _post_body(lo_ref, hi_ref, den_ref, bg_ref,
               w1_ref, b1_ref, w2_ref, b2_ref, w3_ref, b3_ref,
               v1_ref, c1_ref, v2_ref, c2_ref, v3_ref, c3_ref,
               q1_ref, q2_ref):
    acc = jnp.concatenate([lo_ref[...], hi_ref[...]], axis=1)
    g = acc / (den_ref[...] + 1e-16) + bg_ref[...]
    h1 = jax.nn.relu(jnp.dot(g, w1_ref[...], preferred_element_type=jnp.float32)
                     + b1_ref[...])
    h1 = jax.nn.relu(jnp.dot(h1, w2_ref[...], preferred_element_type=jnp.float32)
                     + b2_ref[...])
    q1_ref[...] = jnp.dot(h1, w3_ref[...], preferred_element_type=jnp.float32) + b3_ref[...]
    h2 = jax.nn.relu(jnp.dot(g, v1_ref[...], preferred_element_type=jnp.float32)
                     + c1_ref[...])
    h2 = jax.nn.relu(jnp.dot(h2, v2_ref[...], preferred_element_type=jnp.float32)
                     + c2_ref[...])
    q2_ref[...] = jnp.dot(h2, v3_ref[...], preferred_element_type=jnp.float32) + c3_ref[...]


def _make_w_kernel(np_rows, wins_per_worker, et_pad):
    mesh = plsc.VectorSubcoreMesh(core_axis_name="c", subcore_axis_name="s")

    @functools.partial(
        pl.kernel,
        mesh=mesh,
        compiler_params=_SC_PARAMS,
        out_type=jax.ShapeDtypeStruct((et_pad,), jnp.float32),
        scratch_types=[
            pltpu.VMEM((np_rows,), jnp.float32),
            pltpu.VMEM((_WIN,), jnp.int32),
            pltpu.VMEM((_WIN,), jnp.float32),
            pltpu.VMEM((_WIN,), jnp.float32),
        ],
    )
    def w_kernel(srcp, dstp, asrcp, adstp, ew, tab_v, idx_v, val_v, es_v):
        core = lax.axis_index("c")
        sid = lax.axis_index("s")
        wid = core * _NTILE + sid
        ebase = wid * (wins_per_worker * _WIN)

        pltpu.sync_copy(asrcp, tab_v)

        def p1_body(g, _):
            off = ebase + g * _WIN
            pltpu.sync_copy(srcp.at[pl.ds(off, _WIN)], idx_v)
            for j in range(_WIN // 16):
                sj = idx_v[pl.ds(16 * j, 16)]
                val_v[pl.ds(16 * j, 16)] = plsc.load_gather(tab_v, [sj])
            pltpu.sync_copy(val_v, ew.at[pl.ds(off, _WIN)])
            return 0

        lax.fori_loop(0, wins_per_worker, p1_body, 0)

        pltpu.sync_copy(adstp, tab_v)

        def p2_body(g, _):
            off = ebase + g * _WIN
            pltpu.sync_copy(dstp.at[pl.ds(off, _WIN)], idx_v)
            pltpu.sync_copy(ew.at[pl.ds(off, _WIN)], es_v)
            for j in range(_WIN // 16):
                dj = idx_v[pl.ds(16 * j, 16)]
                ed = plsc.load_gather(tab_v, [dj])
                e = es_v[pl.ds(16 * j, 16)] + ed
                e = jnp.where(e > 0.0, e, 0.2 * e)
                val_v[pl.ds(16 * j, 16)] = jnp.exp(e)
            pltpu.sync_copy(val_v, ew.at[pl.ds(off, _WIN)])
            return 0

        lax.fori_loop(0, wins_per_worker, p2_body, 0)

    return w_kernel


def _make_acc_kernel(np_rows, wins_per_tile, et_pad):
    mesh = plsc.VectorSubcoreMesh(core_axis_name="c", subcore_axis_name="s")
    rows_per_tile = np_rows // _NTILE
    full_chunks = rows_per_tile // _WIN
    tail = rows_per_tile - full_chunks * _WIN

    @functools.partial(
        pl.kernel,
        mesh=mesh,
        compiler_params=_SC_PARAMS,
        out_type=[
            jax.ShapeDtypeStruct((2 * np_rows, 16), jnp.float32),
            jax.ShapeDtypeStruct((2 * np_rows,), jnp.float32),
        ],
        scratch_types=[
            pltpu.VMEM((_WIN,), jnp.int32),
            pltpu.VMEM((_WIN,), jnp.int32),
            pltpu.VMEM((_WIN,), jnp.int32),
            pltpu.VMEM((_WIN,), jnp.float32),
            pltpu.VMEM((_WIN, 16), jnp.float32),
            pltpu.VMEM((_WIN, 16), jnp.float32),
            pltpu.VMEM_SHARED((np_rows, 16), jnp.float32),
            pltpu.VMEM_SHARED((np_rows,), jnp.float32),
            pltpu.SemaphoreType.DMA,
        ],
    )
    def acc_kernel(srcp, dstp, wp, h2, out2, den2,
                   src_v, dst_v, srch_v, w_v, rows_v, epi_v,
                   acc_spm, den_spm, sem):
        core = lax.axis_index("c")
        sid = lax.axis_index("s")
        coff = core * np_rows
        z16 = jnp.zeros((16,), jnp.float32)

        for k in range(_WIN):
            rows_v[k, :] = z16
        for j in range(_WIN // 16):
            w_v[pl.ds(16 * j, 16)] = z16
        rbase = sid * rows_per_tile

        def zero_body(i, _):
            pltpu.sync_copy(rows_v, acc_spm.at[pl.ds(rbase + i * _WIN, _WIN)])
            pltpu.sync_copy(w_v, den_spm.at[pl.ds(rbase + i * _WIN, _WIN)])
            return 0

        lax.fori_loop(0, full_chunks, zero_body, 0)
        if tail:
            pltpu.sync_copy(rows_v.at[pl.ds(0, tail)],
                            acc_spm.at[pl.ds(rbase + full_chunks * _WIN, tail)])
            pltpu.sync_copy(w_v.at[pl.ds(0, tail)],
                            den_spm.at[pl.ds(rbase + full_chunks * _WIN, tail)])
        plsc.subcore_barrier()

        ebase = sid * (wins_per_tile * _WIN)

        def win_body(g, _):
            off = ebase + g * _WIN
            pltpu.sync_copy(srcp.at[pl.ds(off, _WIN)], src_v)
            for j in range(_WIN // 16):
                srch_v[pl.ds(16 * j, 16)] = src_v[pl.ds(16 * j, 16)] + coff
            cp_rows = pltpu.async_copy(h2.at[srch_v], rows_v, sem)
            pltpu.sync_copy(dstp.at[pl.ds(off, _WIN)], dst_v)
            pltpu.sync_copy(wp.at[pl.ds(off, _WIN)], w_v)
            cp_rows.wait()
            for j in range(_WIN // 16):
                wj = w_v[pl.ds(16 * j, 16)]
                for l in range(16):
                    k = 16 * j + l
                    bc = lax.gather(
                        wj, jnp.full((16, 1), l, jnp.int32),
                        lax.GatherDimensionNumbers(
                            offset_dims=(), collapsed_slice_dims=(0,),
                            start_index_map=(0,)),
                        slice_sizes=(1,),
                        mode=lax.GatherScatterMode.PROMISE_IN_BOUNDS)
                    rows_v[k, :] = rows_v[k, :] * bc
            pltpu.sync_copy(rows_v, acc_spm.at[dst_v], add=True)
            pltpu.sync_copy(w_v, den_spm.at[dst_v], add=True)
            return 0

        lax.fori_loop(0, wins_per_tile, win_body, 0)
        plsc.subcore_barrier()

        def epi_body(i, _):
            r0 = sid * rows_per_tile + i * _WIN
            o0 = coff + r0
            pltpu.sync_copy(acc_spm.at[pl.ds(r0, _WIN)], epi_v)
            pltpu.sync_copy(epi_v, out2.at[pl.ds(o0, _WIN)])
            pltpu.sync_copy(den_spm.at[pl.ds(r0, _WIN)], w_v)
            pltpu.sync_copy(w_v, den2.at[pl.ds(o0, _WIN)])
            return 0

        lax.fori_loop(0, full_chunks, epi_body, 0)
        if tail:
            r0 = sid * rows_per_tile + full_chunks * _WIN
            o0 = coff + r0
            pltpu.sync_copy(acc_spm.at[pl.ds(r0, tail)],
                            epi_v.at[pl.ds(0, tail)])
            pltpu.sync_copy(epi_v.at[pl.ds(0, tail)],
                            out2.at[pl.ds(o0, tail)])
            pltpu.sync_copy(den_spm.at[pl.ds(r0, tail)],
                            w_v.at[pl.ds(0, tail)])
            pltpu.sync_copy(w_v.at[pl.ds(0, tail)], den2.at[pl.ds(o0, tail)])

    return acc_kernel


def kernel(s, a, edge_index, W_gat, att_src, att_dst, b_gat,
           W1, b1, W2, b2, W3, b3, V1, c1, V2, c2, V3, c3):
    n, obs = s.shape
    act = a.shape[1]
    hdim = W_gat.shape[1]
    Ws = W_gat[:obs]
    Wa = W_gat[obs:]
    B = 2000
    grid = (n // B,)

    full = lambda shp: pl.BlockSpec(shp, lambda i: tuple(0 for _ in shp))
    hlo, hhi, asrc, adst = pl.pallas_call(
        _pre_body,
        grid=grid,
        in_specs=[
            pl.BlockSpec((B, obs), lambda i: (i, 0)),
            pl.BlockSpec((B, act), lambda i: (i, 0)),
            full((obs, hdim)), full((act, hdim)),
            full((1, hdim)), full((1, hdim)),
        ],
        out_specs=[
            pl.BlockSpec((B, 16), lambda i: (i, 0)),
            pl.BlockSpec((B, 16), lambda i: (i, 0)),
            pl.BlockSpec((B, 1), lambda i: (i, 0)),
            pl.BlockSpec((B, 1), lambda i: (i, 0)),
        ],
        out_shape=[
            jax.ShapeDtypeStruct((n, 16), jnp.float32),
            jax.ShapeDtypeStruct((n, 16), jnp.float32),
            jax.ShapeDtypeStruct((n, 1), jnp.float32),
            jax.ShapeDtypeStruct((n, 1), jnp.float32),
        ],
    )(s, a, Ws, Wa, att_src.reshape(1, hdim), att_dst.reshape(1, hdim))

    et = edge_index.shape[1] + n
    nworker = _NCORE * _NTILE
    per_worker = -(-et // (nworker * _WIN)) * _WIN
    et_pad = per_worker * nworker
    npad = et_pad - et
    np_rows = -(-(n + 16) // (_NTILE * 8)) * (_NTILE * 8)
    wins_per_worker = per_worker // _WIN
    wins_per_tile = 2 * wins_per_worker

    loops = jnp.arange(n, dtype=jnp.int32)
    srcp = jnp.concatenate([edge_index[0], loops,
                            jnp.zeros((npad,), jnp.int32)])
    dstp = jnp.concatenate([edge_index[1], loops,
                            n + (jnp.arange(npad, dtype=jnp.int32) % 16)])
    asrc_p = jnp.pad(asrc.reshape(n), (0, np_rows - n))
    adst_p = jnp.pad(adst.reshape(n), (0, np_rows - n))
    pad16 = ((0, np_rows - n), (0, 0))
    h2 = jnp.concatenate([jnp.pad(hlo, pad16), jnp.pad(hhi, pad16)])

    w_kernel = _make_w_kernel(np_rows, wins_per_worker, et_pad)
    wp = w_kernel(srcp, dstp, asrc_p, adst_p)

    acc_kernel = _make_acc_kernel(np_rows, wins_per_tile, et_pad)
    out2, den2 = acc_kernel(srcp, dstp, wp, h2)

    lo = out2[:n]
    hi = out2[np_rows:np_rows + n]
    den = den2[:n].reshape(n, 1)

    q1, q2 = pl.pallas_call(
        _post_body,
        grid=grid,
        in_specs=[
            pl.BlockSpec((B, 16), lambda i: (i, 0)),
            pl.BlockSpec((B, 16), lambda i: (i, 0)),
            pl.BlockSpec((B, 1), lambda i: (i, 0)),
            full((1, hdim)),
            full((hdim, hdim)), full((1, hdim)),
            full((hdim, hdim)), full((1, hdim)),
            full((hdim, 1)), full((1, 1)),
            full((hdim, hdim)), full((1, hdim)),
            full((hdim, hdim)), full((1, hdim)),
            full((hdim, 1)), full((1, 1)),
        ],
        out_specs=[
            pl.BlockSpec((B, 1), lambda i: (i, 0)),
            pl.BlockSpec((B, 1), lambda i: (i, 0)),
        ],
        out_shape=[
            jax.ShapeDtypeStruct((n, 1), jnp.float32),
            jax.ShapeDtypeStruct((n, 1), jnp.float32),
        ],
    )(lo, hi, den, b_gat.reshape(1, hdim),
      W1, b1.reshape(1, hdim), W2, b2.reshape(1, hdim), W3, b3.reshape(1, 1),
      V1, c1.reshape(1, hdim), V2, c2.reshape(1, hdim), V3, c3.reshape(1, 1))
    return (q1, q2)


# speedup vs baseline: 22.2153x; 1.1144x over previous
import functools

import jax
import jax.numpy as jnp
from jax import lax
from jax.experimental import pallas as pl
from jax.experimental.pallas import tpu as pltpu
from jax.experimental.pallas import tpu_sc as plsc

_WIN = 128
_NTILE = 16
_NCORE = 2
_SC_PARAMS = pltpu.CompilerParams(use_tc_tiling_on_sc=False,
                                  needs_layout_passes=False)


def _pre_body(s_ref, a_ref, ws_ref, wa_ref, atts_ref, attd_ref,
              hlo_ref, hhi_ref, asrc_ref, adst_ref):
    hb = (jnp.dot(s_ref[...], ws_ref[...], preferred_element_type=jnp.float32)
          + jnp.dot(a_ref[...], wa_ref[...], preferred_element_type=jnp.float32))
    hlo_ref[...] = hb[:, :16]
    hhi_ref[...] = hb[:, 16:]
    asrc_ref[...] = jnp.sum(hb * atts_ref[...], axis=1, keepdims=True)
    adst_ref[...] = jnp.sum(hb * attd_ref[...], axis=1, keepdims=True)


def _post_body(lo_ref, hi_ref, den_ref, bg_ref,
               w1_ref, b1_ref, w2_ref, b2_ref, w3_ref, b3_ref,
               v1_ref, c1_ref, v2_ref, c2_ref, v3_ref, c3_ref,
               q1_ref, q2_ref):
    acc = jnp.concatenate([lo_ref[...], hi_ref[...]], axis=1)
    g = acc / (den_ref[...] + 1e-16) + bg_ref[...]
    h1 = jax.nn.relu(jnp.dot(g, w1_ref[...], preferred_element_type=jnp.float32)
                     + b1_ref[...])
    h1 = jax.nn.relu(jnp.dot(h1, w2_ref[...], preferred_element_type=jnp.float32)
                     + b2_ref[...])
    q1_ref[...] = jnp.dot(h1, w3_ref[...], preferred_element_type=jnp.float32) + b3_ref[...]
    h2 = jax.nn.relu(jnp.dot(g, v1_ref[...], preferred_element_type=jnp.float32)
                     + c1_ref[...])
    h2 = jax.nn.relu(jnp.dot(h2, v2_ref[...], preferred_element_type=jnp.float32)
                     + c2_ref[...])
    q2_ref[...] = jnp.dot(h2, v3_ref[...], preferred_element_type=jnp.float32) + c3_ref[...]


def _make_w_kernel(np_rows, wins_per_worker, et_pad):
    mesh = plsc.VectorSubcoreMesh(core_axis_name="c", subcore_axis_name="s")

    @functools.partial(
        pl.kernel,
        mesh=mesh,
        compiler_params=_SC_PARAMS,
        out_type=jax.ShapeDtypeStruct((et_pad,), jnp.float32),
        scratch_types=[
            pltpu.VMEM((np_rows,), jnp.float32),
            pltpu.VMEM((_WIN,), jnp.int32),
            pltpu.VMEM((_WIN,), jnp.float32),
            pltpu.VMEM((_WIN,), jnp.float32),
        ],
    )
    def w_kernel(srcp, dstp, asrcp, adstp, ew, tab_v, idx_v, val_v, es_v):
        core = lax.axis_index("c")
        sid = lax.axis_index("s")
        wid = core * _NTILE + sid
        ebase = wid * (wins_per_worker * _WIN)

        pltpu.sync_copy(asrcp, tab_v)

        def p1_body(g, _):
            off = ebase + g * _WIN
            pltpu.sync_copy(srcp.at[pl.ds(off, _WIN)], idx_v)
            for j in range(_WIN // 16):
                sj = idx_v[pl.ds(16 * j, 16)]
                val_v[pl.ds(16 * j, 16)] = plsc.load_gather(tab_v, [sj])
            pltpu.sync_copy(val_v, ew.at[pl.ds(off, _WIN)])
            return 0

        lax.fori_loop(0, wins_per_worker, p1_body, 0)

        pltpu.sync_copy(adstp, tab_v)

        def p2_body(g, _):
            off = ebase + g * _WIN
            pltpu.sync_copy(dstp.at[pl.ds(off, _WIN)], idx_v)
            pltpu.sync_copy(ew.at[pl.ds(off, _WIN)], es_v)
            for j in range(_WIN // 16):
                dj = idx_v[pl.ds(16 * j, 16)]
                ed = plsc.load_gather(tab_v, [dj])
                e = es_v[pl.ds(16 * j, 16)] + ed
                e = jnp.where(e > 0.0, e, 0.2 * e)
                val_v[pl.ds(16 * j, 16)] = jnp.exp(e)
            pltpu.sync_copy(val_v, ew.at[pl.ds(off, _WIN)])
            return 0

        lax.fori_loop(0, wins_per_worker, p2_body, 0)

    return w_kernel


def _make_acc_kernel(np_rows, wins_per_tile, et_pad):
    mesh = plsc.VectorSubcoreMesh(core_axis_name="c", subcore_axis_name="s")
    rows_per_tile = np_rows // _NTILE
    full_chunks = rows_per_tile // _WIN
    tail = rows_per_tile - full_chunks * _WIN

    @functools.partial(
        pl.kernel,
        mesh=mesh,
        compiler_params=_SC_PARAMS,
        out_type=[
            jax.ShapeDtypeStruct((2 * np_rows, 16), jnp.float32),
            jax.ShapeDtypeStruct((2 * np_rows,), jnp.float32),
        ],
        scratch_types=[
            [pltpu.VMEM((_WIN,), jnp.int32)] * 2,
            [pltpu.VMEM((_WIN,), jnp.int32)] * 2,
            [pltpu.VMEM((_WIN,), jnp.int32)] * 2,
            [pltpu.VMEM((_WIN,), jnp.float32)] * 2,
            [pltpu.VMEM((_WIN, 16), jnp.float32)] * 2,
            pltpu.VMEM((_WIN, 16), jnp.float32),
            pltpu.VMEM_SHARED((np_rows, 16), jnp.float32),
            pltpu.VMEM_SHARED((np_rows,), jnp.float32),
            [pltpu.SemaphoreType.DMA] * 2,
            [pltpu.SemaphoreType.DMA] * 2,
            [pltpu.SemaphoreType.DMA] * 2,
        ],
    )
    def acc_kernel(srcp, dstp, wp, h2, out2, den2,
                   src_v, dst_v, srch_v, w_v, rows_v, epi_v,
                   acc_spm, den_spm, sem_ld, sem_g, sem_sc):
        core = lax.axis_index("c")
        sid = lax.axis_index("s")
        coff = core * np_rows
        z16 = jnp.zeros((16,), jnp.float32)
        mk = pltpu.make_async_copy

        def scale_window(b):
            for j in range(_WIN // 16):
                wj = w_v[b][pl.ds(16 * j, 16)]
                for l in range(16):
                    k = 16 * j + l
                    bc = lax.gather(
                        wj, jnp.full((16, 1), l, jnp.int32),
                        lax.GatherDimensionNumbers(
                            offset_dims=(), collapsed_slice_dims=(0,),
                            start_index_map=(0,)),
                        slice_sizes=(1,),
                        mode=lax.GatherScatterMode.PROMISE_IN_BOUNDS)
                    rows_v[b][k, :] = rows_v[b][k, :] * bc

        def issue_loads(bb, off2):
            pltpu.async_copy(srcp.at[pl.ds(off2, _WIN)], src_v[bb], sem_ld[bb])
            pltpu.async_copy(dstp.at[pl.ds(off2, _WIN)], dst_v[bb], sem_ld[bb])
            pltpu.async_copy(wp.at[pl.ds(off2, _WIN)], w_v[bb], sem_ld[bb])

        def wait_loads(bb, off2):
            mk(srcp.at[pl.ds(off2, _WIN)], src_v[bb], sem_ld[bb]).wait()
            mk(dstp.at[pl.ds(off2, _WIN)], dst_v[bb], sem_ld[bb]).wait()
            mk(wp.at[pl.ds(off2, _WIN)], w_v[bb], sem_ld[bb]).wait()

        def issue_gather(bb):
            for j in range(_WIN // 16):
                srch_v[bb][pl.ds(16 * j, 16)] = (src_v[bb][pl.ds(16 * j, 16)]
                                                 + coff)
            pltpu.async_copy(h2.at[srch_v[bb]], rows_v[bb], sem_g[bb])

        def issue_scatter(b):
            pltpu.async_copy(rows_v[b], acc_spm.at[dst_v[b]], sem_sc[b],
                             add=True)
            pltpu.async_copy(w_v[b], den_spm.at[dst_v[b]], sem_sc[b],
                             add=True)

        def wait_scatter(b):
            mk(rows_v[b], acc_spm.at[dst_v[b]], sem_sc[b]).wait()
            mk(w_v[b], den_spm.at[dst_v[b]], sem_sc[b]).wait()

        for b in range(2):
            for k in range(_WIN):
                rows_v[b][k, :] = z16
            for j in range(_WIN // 16):
                w_v[b][pl.ds(16 * j, 16)] = z16
                dst_v[b][pl.ds(16 * j, 16)] = (
                    jnp.full((16,), np_rows - 16, jnp.int32)
                    + jax.lax.iota(jnp.int32, 16))
        rbase = sid * rows_per_tile

        def zero_body(i, _):
            pltpu.sync_copy(rows_v[0],
                            acc_spm.at[pl.ds(rbase + i * _WIN, _WIN)])
            pltpu.sync_copy(w_v[0], den_spm.at[pl.ds(rbase + i * _WIN, _WIN)])
            return 0

        lax.fori_loop(0, full_chunks, zero_body, 0)
        if tail:
            pltpu.sync_copy(rows_v[0].at[pl.ds(0, tail)],
                            acc_spm.at[pl.ds(rbase + full_chunks * _WIN, tail)])
            pltpu.sync_copy(w_v[0].at[pl.ds(0, tail)],
                            den_spm.at[pl.ds(rbase + full_chunks * _WIN, tail)])
        plsc.subcore_barrier()

        ebase = sid * (wins_per_tile * _WIN)
        issue_scatter(1)
        pltpu.sync_copy(srcp.at[pl.ds(ebase, _WIN)], src_v[0])
        pltpu.sync_copy(dstp.at[pl.ds(ebase, _WIN)], dst_v[0])
        pltpu.sync_copy(wp.at[pl.ds(ebase, _WIN)], w_v[0])
        issue_gather(0)

        def win_body(i, _):
            for b in range(2):
                bb = 1 - b
                g = 2 * i + b
                off2 = ebase + (g + 1) * _WIN
                mk(h2.at[srch_v[b]], rows_v[b], sem_g[b]).wait()
                scale_window(b)
                wait_scatter(bb)
                issue_loads(bb, off2)
                issue_scatter(b)
                wait_loads(bb, off2)
                issue_gather(bb)
            return 0

        lax.fori_loop(0, wins_per_tile // 2, win_body, 0)
        mk(h2.at[srch_v[0]], rows_v[0], sem_g[0]).wait()
        wait_scatter(1)
        plsc.subcore_barrier()

        def epi_body(i, _):
            r0 = sid * rows_per_tile + i * _WIN
            o0 = coff + r0
            pltpu.sync_copy(acc_spm.at[pl.ds(r0, _WIN)], epi_v)
            pltpu.sync_copy(epi_v, out2.at[pl.ds(o0, _WIN)])
            pltpu.sync_copy(den_spm.at[pl.ds(r0, _WIN)], w_v[0])
            pltpu.sync_copy(w_v[0], den2.at[pl.ds(o0, _WIN)])
            return 0

        lax.fori_loop(0, full_chunks, epi_body, 0)
        if tail:
            r0 = sid * rows_per_tile + full_chunks * _WIN
            o0 = coff + r0
            pltpu.sync_copy(acc_spm.at[pl.ds(r0, tail)],
                            epi_v.at[pl.ds(0, tail)])
            pltpu.sync_copy(epi_v.at[pl.ds(0, tail)],
                            out2.at[pl.ds(o0, tail)])
            pltpu.sync_copy(den_spm.at[pl.ds(r0, tail)],
                            w_v[0].at[pl.ds(0, tail)])
            pltpu.sync_copy(w_v[0].at[pl.ds(0, tail)],
                            den2.at[pl.ds(o0, tail)])

    return acc_kernel


def kernel(s, a, edge_index, W_gat, att_src, att_dst, b_gat,
           W1, b1, W2, b2, W3, b3, V1, c1, V2, c2, V3, c3):
    n, obs = s.shape
    act = a.shape[1]
    hdim = W_gat.shape[1]
    Ws = W_gat[:obs]
    Wa = W_gat[obs:]
    B = 2000
    grid = (n // B,)

    full = lambda shp: pl.BlockSpec(shp, lambda i: tuple(0 for _ in shp))
    hlo, hhi, asrc, adst = pl.pallas_call(
        _pre_body,
        grid=grid,
        in_specs=[
            pl.BlockSpec((B, obs), lambda i: (i, 0)),
            pl.BlockSpec((B, act), lambda i: (i, 0)),
            full((obs, hdim)), full((act, hdim)),
            full((1, hdim)), full((1, hdim)),
        ],
        out_specs=[
            pl.BlockSpec((B, 16), lambda i: (i, 0)),
            pl.BlockSpec((B, 16), lambda i: (i, 0)),
            pl.BlockSpec((B, 1), lambda i: (i, 0)),
            pl.BlockSpec((B, 1), lambda i: (i, 0)),
        ],
        out_shape=[
            jax.ShapeDtypeStruct((n, 16), jnp.float32),
            jax.ShapeDtypeStruct((n, 16), jnp.float32),
            jax.ShapeDtypeStruct((n, 1), jnp.float32),
            jax.ShapeDtypeStruct((n, 1), jnp.float32),
        ],
    )(s, a, Ws, Wa, att_src.reshape(1, hdim), att_dst.reshape(1, hdim))

    et = edge_index.shape[1] + n
    nworker = _NCORE * _NTILE
    per_worker = -(-et // (nworker * _WIN)) * _WIN
    et_pad = per_worker * nworker
    et_alloc = et_pad + nworker * _WIN
    npad = et_alloc - et
    np_rows = -(-(n + 16) // (_NTILE * 8)) * (_NTILE * 8)
    wins_per_worker = per_worker // _WIN + 1
    wins_per_tile = 2 * (per_worker // _WIN)

    loops = jnp.arange(n, dtype=jnp.int32)
    srcp = jnp.concatenate([edge_index[0], loops,
                            jnp.zeros((npad,), jnp.int32)])
    dstp = jnp.concatenate([edge_index[1], loops,
                            n + (jnp.arange(npad, dtype=jnp.int32) % 16)])
    asrc_p = jnp.pad(asrc.reshape(n), (0, np_rows - n))
    adst_p = jnp.pad(adst.reshape(n), (0, np_rows - n))
    pad16 = ((0, np_rows - n), (0, 0))
    h2 = jnp.concatenate([jnp.pad(hlo, pad16), jnp.pad(hhi, pad16)])

    w_kernel = _make_w_kernel(np_rows, wins_per_worker, et_alloc)
    wp = w_kernel(srcp, dstp, asrc_p, adst_p)

    acc_kernel = _make_acc_kernel(np_rows, wins_per_tile, et_alloc)
    out2, den2 = acc_kernel(srcp, dstp, wp, h2)

    lo = out2[:n]
    hi = out2[np_rows:np_rows + n]
    den = den2[:n].reshape(n, 1)

    q1, q2 = pl.pallas_call(
        _post_body,
        grid=grid,
        in_specs=[
            pl.BlockSpec((B, 16), lambda i: (i, 0)),
            pl.BlockSpec((B, 16), lambda i: (i, 0)),
            pl.BlockSpec((B, 1), lambda i: (i, 0)),
            full((1, hdim)),
            full((hdim, hdim)), full((1, hdim)),
            full((hdim, hdim)), full((1, hdim)),
            full((hdim, 1)), full((1, 1)),
            full((hdim, hdim)), full((1, hdim)),
            full((hdim, hdim)), full((1, hdim)),
            full((hdim, 1)), full((1, 1)),
        ],
        out_specs=[
            pl.BlockSpec((B, 1), lambda i: (i, 0)),
            pl.BlockSpec((B, 1), lambda i: (i, 0)),
        ],
        out_shape=[
            jax.ShapeDtypeStruct((n, 1), jnp.float32),
            jax.ShapeDtypeStruct((n, 1), jnp.float32),
        ],
    )(lo, hi, den, b_gat.reshape(1, hdim),
      W1, b1.reshape(1, hdim), W2, b2.reshape(1, hdim), W3, b3.reshape(1, 1),
      V1, c1.reshape(1, hdim), V2, c2.reshape(1, hdim), V3, c3.reshape(1, 1))
    return (q1, q2)


# speedup vs baseline: 23.8823x; 1.0750x over previous
import functools

import jax
import jax.numpy as jnp
from jax import lax
from jax.experimental import pallas as pl
from jax.experimental.pallas import tpu as pltpu
from jax.experimental.pallas import tpu_sc as plsc

_WIN = 128
_NTILE = 16
_NCORE = 2
_SC_PARAMS = pltpu.CompilerParams(use_tc_tiling_on_sc=False,
                                  needs_layout_passes=False)


def _pre_body(s_ref, a_ref, ws_ref, wa_ref, atts_ref, attd_ref,
              hlo_ref, hhi_ref, asrc_ref, adst_ref):
    hb = (jnp.dot(s_ref[...], ws_ref[...], preferred_element_type=jnp.float32)
          + jnp.dot(a_ref[...], wa_ref[...], preferred_element_type=jnp.float32))
    hlo_ref[...] = hb[:, :16]
    hhi_ref[...] = hb[:, 16:]
    asrc_ref[...] = jnp.sum(hb * atts_ref[...], axis=1, keepdims=True)
    adst_ref[...] = jnp.sum(hb * attd_ref[...], axis=1, keepdims=True)


def _post_body(lo_ref, hi_ref, den_ref, bg_ref,
               w1_ref, b1_ref, w2_ref, b2_ref, w3_ref, b3_ref,
               v1_ref, c1_ref, v2_ref, c2_ref, v3_ref, c3_ref,
               q1_ref, q2_ref):
    acc = jnp.concatenate([lo_ref[...], hi_ref[...]], axis=1)
    g = acc / (den_ref[...] + 1e-16) + bg_ref[...]
    h1 = jax.nn.relu(jnp.dot(g, w1_ref[...], preferred_element_type=jnp.float32)
                     + b1_ref[...])
    h1 = jax.nn.relu(jnp.dot(h1, w2_ref[...], preferred_element_type=jnp.float32)
                     + b2_ref[...])
    q1_ref[...] = jnp.dot(h1, w3_ref[...], preferred_element_type=jnp.float32) + b3_ref[...]
    h2 = jax.nn.relu(jnp.dot(g, v1_ref[...], preferred_element_type=jnp.float32)
                     + c1_ref[...])
    h2 = jax.nn.relu(jnp.dot(h2, v2_ref[...], preferred_element_type=jnp.float32)
                     + c2_ref[...])
    q2_ref[...] = jnp.dot(h2, v3_ref[...], preferred_element_type=jnp.float32) + c3_ref[...]


def _make_acc_kernel(np_rows, wins_per_tile):
    mesh = plsc.VectorSubcoreMesh(core_axis_name="c", subcore_axis_name="s")
    rows_per_tile = np_rows // _NTILE
    full_chunks = rows_per_tile // _WIN
    tail = rows_per_tile - full_chunks * _WIN

    @functools.partial(
        pl.kernel,
        mesh=mesh,
        compiler_params=_SC_PARAMS,
        out_type=[
            jax.ShapeDtypeStruct((2 * np_rows, 16), jnp.float32),
            jax.ShapeDtypeStruct((2 * np_rows,), jnp.float32),
        ],
        scratch_types=[
            [pltpu.VMEM((_WIN,), jnp.int32)] * 2,
            [pltpu.VMEM((_WIN,), jnp.int32)] * 2,
            [pltpu.VMEM((_WIN,), jnp.int32)] * 2,
            [pltpu.VMEM((_WIN, 32), jnp.float32)] * 2,
            [pltpu.VMEM((_WIN, 16), jnp.float32)] * 2,
            [pltpu.VMEM((_WIN, 16), jnp.float32)] * 2,
            [pltpu.VMEM((_WIN,), jnp.float32)] * 2,
            pltpu.VMEM((_WIN, 16), jnp.float32),
            pltpu.VMEM_SHARED((np_rows, 16), jnp.float32),
            pltpu.VMEM_SHARED((np_rows,), jnp.float32),
            [pltpu.SemaphoreType.DMA] * 2,
            [pltpu.SemaphoreType.DMA] * 2,
            [pltpu.SemaphoreType.DMA] * 2,
        ],
    )
    def acc_kernel(srcp, dstp, adst_tab, h2a, out2, den2,
                   src_v, dst_v, srch_v, rows_v, dd_v, sc_rows, w_sc, epi_v,
                   acc_spm, den_spm, sem_ld, sem_g, sem_sc):
        core = lax.axis_index("c")
        sid = lax.axis_index("s")
        coff = core * np_rows
        z16 = jnp.zeros((16,), jnp.float32)
        i16 = lax.iota(jnp.int32, 16)
        mk = pltpu.make_async_copy
        gdn = lax.GatherDimensionNumbers(
            offset_dims=(), collapsed_slice_dims=(0,), start_index_map=(0,))

        def compute_window(b):
            for j in range(_WIN // 16):
                es = plsc.load_gather(rows_v[b],
                                      [16 * j + i16, jnp.full((16,), 16,
                                                              jnp.int32)])
                ed = plsc.load_gather(dd_v[b],
                                      [16 * j + i16,
                                       jnp.zeros((16,), jnp.int32)])
                e = es + ed
                e = jnp.where(e > 0.0, e, 0.2 * e)
                wj = jnp.exp(e)
                w_sc[b][pl.ds(16 * j, 16)] = wj
                for l in range(16):
                    k = 16 * j + l
                    bc = lax.gather(
                        wj, jnp.full((16, 1), l, jnp.int32), gdn,
                        slice_sizes=(1,),
                        mode=lax.GatherScatterMode.PROMISE_IN_BOUNDS)
                    sc_rows[b][k, :] = rows_v[b][k, pl.ds(0, 16)] * bc

        def issue_loads(bb, off2):
            pltpu.async_copy(srcp.at[pl.ds(off2, _WIN)], src_v[bb], sem_ld[bb])
            pltpu.async_copy(dstp.at[pl.ds(off2, _WIN)], dst_v[bb], sem_ld[bb])

        def wait_loads(bb, off2):
            mk(srcp.at[pl.ds(off2, _WIN)], src_v[bb], sem_ld[bb]).wait()
            mk(dstp.at[pl.ds(off2, _WIN)], dst_v[bb], sem_ld[bb]).wait()

        def issue_gather(bb):
            for j in range(_WIN // 16):
                srch_v[bb][pl.ds(16 * j, 16)] = (src_v[bb][pl.ds(16 * j, 16)]
                                                 + coff)
            pltpu.async_copy(h2a.at[srch_v[bb]], rows_v[bb], sem_g[bb])
            pltpu.async_copy(adst_tab.at[dst_v[bb]], dd_v[bb], sem_g[bb])

        def wait_gather(b):
            mk(h2a.at[srch_v[b]], rows_v[b], sem_g[b]).wait()
            mk(adst_tab.at[dst_v[b]], dd_v[b], sem_g[b]).wait()

        def issue_scatter(b):
            pltpu.async_copy(sc_rows[b], acc_spm.at[dst_v[b]], sem_sc[b],
                             add=True)
            pltpu.async_copy(w_sc[b], den_spm.at[dst_v[b]], sem_sc[b],
                             add=True)

        def wait_scatter(b):
            mk(sc_rows[b], acc_spm.at[dst_v[b]], sem_sc[b]).wait()
            mk(w_sc[b], den_spm.at[dst_v[b]], sem_sc[b]).wait()

        for b in range(2):
            for k in range(_WIN):
                sc_rows[b][k, :] = z16
            for j in range(_WIN // 16):
                w_sc[b][pl.ds(16 * j, 16)] = z16
                dst_v[b][pl.ds(16 * j, 16)] = (
                    jnp.full((16,), np_rows - 16, jnp.int32) + i16)
        rbase = sid * rows_per_tile

        def zero_body(i, _):
            pltpu.sync_copy(sc_rows[0],
                            acc_spm.at[pl.ds(rbase + i * _WIN, _WIN)])
            pltpu.sync_copy(w_sc[0], den_spm.at[pl.ds(rbase + i * _WIN, _WIN)])
            return 0

        lax.fori_loop(0, full_chunks, zero_body, 0)
        if tail:
            pltpu.sync_copy(sc_rows[0].at[pl.ds(0, tail)],
                            acc_spm.at[pl.ds(rbase + full_chunks * _WIN, tail)])
            pltpu.sync_copy(w_sc[0].at[pl.ds(0, tail)],
                            den_spm.at[pl.ds(rbase + full_chunks * _WIN, tail)])
        plsc.subcore_barrier()

        ebase = sid * (wins_per_tile * _WIN)
        issue_scatter(1)
        pltpu.sync_copy(srcp.at[pl.ds(ebase, _WIN)], src_v[0])
        pltpu.sync_copy(dstp.at[pl.ds(ebase, _WIN)], dst_v[0])
        issue_gather(0)

        def win_body(i, _):
            for b in range(2):
                bb = 1 - b
                g = 2 * i + b
                off2 = ebase + (g + 1) * _WIN
                wait_gather(b)
                compute_window(b)
                wait_scatter(bb)
                issue_loads(bb, off2)
                issue_scatter(b)
                wait_loads(bb, off2)
                issue_gather(bb)
            return 0

        lax.fori_loop(0, wins_per_tile // 2, win_body, 0)
        wait_gather(0)
        wait_scatter(1)
        plsc.subcore_barrier()

        def epi_body(i, _):
            r0 = sid * rows_per_tile + i * _WIN
            o0 = coff + r0
            pltpu.sync_copy(acc_spm.at[pl.ds(r0, _WIN)], epi_v)
            pltpu.sync_copy(epi_v, out2.at[pl.ds(o0, _WIN)])
            pltpu.sync_copy(den_spm.at[pl.ds(r0, _WIN)], w_sc[0])
            pltpu.sync_copy(w_sc[0], den2.at[pl.ds(o0, _WIN)])
            return 0

        lax.fori_loop(0, full_chunks, epi_body, 0)
        if tail:
            r0 = sid * rows_per_tile + full_chunks * _WIN
            o0 = coff + r0
            pltpu.sync_copy(acc_spm.at[pl.ds(r0, tail)],
                            epi_v.at[pl.ds(0, tail)])
            pltpu.sync_copy(epi_v.at[pl.ds(0, tail)],
                            out2.at[pl.ds(o0, tail)])
            pltpu.sync_copy(den_spm.at[pl.ds(r0, tail)],
                            w_sc[0].at[pl.ds(0, tail)])
            pltpu.sync_copy(w_sc[0].at[pl.ds(0, tail)],
                            den2.at[pl.ds(o0, tail)])

    return acc_kernel


def kernel(s, a, edge_index, W_gat, att_src, att_dst, b_gat,
           W1, b1, W2, b2, W3, b3, V1, c1, V2, c2, V3, c3):
    n, obs = s.shape
    act = a.shape[1]
    hdim = W_gat.shape[1]
    Ws = W_gat[:obs]
    Wa = W_gat[obs:]
    B = 2000
    grid = (n // B,)

    full = lambda shp: pl.BlockSpec(shp, lambda i: tuple(0 for _ in shp))
    hlo, hhi, asrc, adst = pl.pallas_call(
        _pre_body,
        grid=grid,
        in_specs=[
            pl.BlockSpec((B, obs), lambda i: (i, 0)),
            pl.BlockSpec((B, act), lambda i: (i, 0)),
            full((obs, hdim)), full((act, hdim)),
            full((1, hdim)), full((1, hdim)),
        ],
        out_specs=[
            pl.BlockSpec((B, 16), lambda i: (i, 0)),
            pl.BlockSpec((B, 16), lambda i: (i, 0)),
            pl.BlockSpec((B, 1), lambda i: (i, 0)),
            pl.BlockSpec((B, 1), lambda i: (i, 0)),
        ],
        out_shape=[
            jax.ShapeDtypeStruct((n, 16), jnp.float32),
            jax.ShapeDtypeStruct((n, 16), jnp.float32),
            jax.ShapeDtypeStruct((n, 1), jnp.float32),
            jax.ShapeDtypeStruct((n, 1), jnp.float32),
        ],
    )(s, a, Ws, Wa, att_src.reshape(1, hdim), att_dst.reshape(1, hdim))

    et = edge_index.shape[1] + n
    nworker = _NCORE * _NTILE
    per_worker = -(-et // (nworker * _WIN)) * _WIN
    et_pad = per_worker * nworker
    et_alloc = et_pad + _NTILE * _WIN
    npad = et_alloc - et
    np_rows = -(-(n + 16) // (_NTILE * 8)) * (_NTILE * 8)
    wins_per_tile = 2 * (per_worker // _WIN)

    loops = jnp.arange(n, dtype=jnp.int32)
    srcp = jnp.concatenate([edge_index[0], loops,
                            jnp.zeros((npad,), jnp.int32)])
    dstp = jnp.concatenate([edge_index[1], loops,
                            n + (jnp.arange(npad, dtype=jnp.int32) % 16)])
    asrc_p = jnp.pad(asrc.reshape(n), (0, np_rows - n))
    adst_p = jnp.pad(adst.reshape(n), (0, np_rows - n))
    pad16 = ((0, np_rows - n), (0, 0))
    h2 = jnp.concatenate([jnp.pad(hlo, pad16), jnp.pad(hhi, pad16)])
    asrc2 = jnp.concatenate([asrc_p, asrc_p]).reshape(2 * np_rows, 1)
    h2a = jnp.concatenate(
        [h2, asrc2, jnp.zeros((2 * np_rows, 15), jnp.float32)], axis=1)
    adst_tab = jnp.concatenate(
        [adst_p.reshape(np_rows, 1), jnp.zeros((np_rows, 15), jnp.float32)],
        axis=1)

    acc_kernel = _make_acc_kernel(np_rows, wins_per_tile)
    out2, den2 = acc_kernel(srcp, dstp, adst_tab, h2a)

    lo = out2[:n]
    hi = out2[np_rows:np_rows + n]
    den = den2[:n].reshape(n, 1)

    q1, q2 = pl.pallas_call(
        _post_body,
        grid=grid,
        in_specs=[
            pl.BlockSpec((B, 16), lambda i: (i, 0)),
            pl.BlockSpec((B, 16), lambda i: (i, 0)),
            pl.BlockSpec((B, 1), lambda i: (i, 0)),
            full((1, hdim)),
            full((hdim, hdim)), full((1, hdim)),
            full((hdim, hdim)), full((1, hdim)),
            full((hdim, 1)), full((1, 1)),
            full((hdim, hdim)), full((1, hdim)),
            full((hdim, hdim)), full((1, hdim)),
            full((hdim, 1)), full((1, 1)),
        ],
        out_specs=[
            pl.BlockSpec((B, 1), lambda i: (i, 0)),
            pl.BlockSpec((B, 1), lambda i: (i, 0)),
        ],
        out_shape=[
            jax.ShapeDtypeStruct((n, 1), jnp.float32),
            jax.ShapeDtypeStruct((n, 1), jnp.float32),
        ],
    )(lo, hi, den, b_gat.reshape(1, hdim),
      W1, b1.reshape(1, hdim), W2, b2.reshape(1, hdim), W3, b3.reshape(1, 1),
      V1, c1.reshape(1, hdim), V2, c2.reshape(1, hdim), V3, c3.reshape(1, 1))
    return (q1, q2)


# speedup vs baseline: 27.4336x; 1.1487x over previous
import functools

import jax
import jax.numpy as jnp
from jax import lax
from jax.experimental import pallas as pl
from jax.experimental.pallas import tpu as pltpu
from jax.experimental.pallas import tpu_sc as plsc

_WIN = 112
_NTILE = 16
_NCORE = 2
_SC_PARAMS = pltpu.CompilerParams(use_tc_tiling_on_sc=False,
                                  needs_layout_passes=False)


def _pre_body(s_ref, a_ref, ws_ref, wa_ref, atts_ref, attd_ref,
              hlo_ref, hhi_ref, asrc_ref, adst_ref):
    hb = (jnp.dot(s_ref[...], ws_ref[...], preferred_element_type=jnp.float32)
          + jnp.dot(a_ref[...], wa_ref[...], preferred_element_type=jnp.float32))
    hlo_ref[...] = hb[:, :16]
    hhi_ref[...] = hb[:, 16:]
    asrc_ref[...] = jnp.sum(hb * atts_ref[...], axis=1, keepdims=True)
    adst_ref[...] = jnp.sum(hb * attd_ref[...], axis=1, keepdims=True)


def _post_body(lo_ref, hi_ref, den_ref, bg_ref,
               w1_ref, b1_ref, w2_ref, b2_ref, w3_ref, b3_ref,
               v1_ref, c1_ref, v2_ref, c2_ref, v3_ref, c3_ref,
               q1_ref, q2_ref):
    acc = jnp.concatenate([lo_ref[...], hi_ref[...]], axis=1)
    g = acc / (den_ref[...] + 1e-16) + bg_ref[...]
    h1 = jax.nn.relu(jnp.dot(g, w1_ref[...], preferred_element_type=jnp.float32)
                     + b1_ref[...])
    h1 = jax.nn.relu(jnp.dot(h1, w2_ref[...], preferred_element_type=jnp.float32)
                     + b2_ref[...])
    q1_ref[...] = jnp.dot(h1, w3_ref[...], preferred_element_type=jnp.float32) + b3_ref[...]
    h2 = jax.nn.relu(jnp.dot(g, v1_ref[...], preferred_element_type=jnp.float32)
                     + c1_ref[...])
    h2 = jax.nn.relu(jnp.dot(h2, v2_ref[...], preferred_element_type=jnp.float32)
                     + c2_ref[...])
    q2_ref[...] = jnp.dot(h2, v3_ref[...], preferred_element_type=jnp.float32) + c3_ref[...]


def _make_acc_kernel(np_rows, wins_per_tile):
    mesh = plsc.VectorSubcoreMesh(core_axis_name="c", subcore_axis_name="s")
    rows_per_tile = np_rows // _NTILE
    full_chunks = rows_per_tile // _WIN
    tail = rows_per_tile - full_chunks * _WIN

    @functools.partial(
        pl.kernel,
        mesh=mesh,
        compiler_params=_SC_PARAMS,
        out_type=[
            jax.ShapeDtypeStruct((2 * np_rows, 16), jnp.float32),
            jax.ShapeDtypeStruct((2 * np_rows,), jnp.float32),
        ],
        scratch_types=[
            [pltpu.VMEM((_WIN,), jnp.int32)] * 3,
            [pltpu.VMEM((_WIN,), jnp.int32)] * 3,
            [pltpu.VMEM((_WIN,), jnp.int32)] * 3,
            [pltpu.VMEM((_WIN, 32), jnp.float32)] * 3,
            [pltpu.VMEM((_WIN, 16), jnp.float32)] * 3,
            [pltpu.VMEM((_WIN, 16), jnp.float32)] * 3,
            [pltpu.VMEM((_WIN,), jnp.float32)] * 3,
            [pltpu.VMEM((_WIN,), jnp.int32)] * 3,
            pltpu.VMEM_SHARED((np_rows, 16), jnp.float32),
            pltpu.VMEM_SHARED((np_rows,), jnp.float32),
            [pltpu.SemaphoreType.DMA] * 3,
            [pltpu.SemaphoreType.DMA] * 3,
            [pltpu.SemaphoreType.DMA] * 3,
        ],
    )
    def acc_kernel(srcp, dstp, adst_tab, h2a, out2, den2,
                   src_v, dst_v, srch_v, rows_v, dd_v, sc_rows, w_sc,
                   dstsc_v, acc_spm, den_spm, sem_ld, sem_g, sem_sc):
        core = lax.axis_index("c")
        sid = lax.axis_index("s")
        coff = core * np_rows
        z16 = jnp.zeros((16,), jnp.float32)
        i16 = lax.iota(jnp.int32, 16)
        mk = pltpu.make_async_copy
        gdn = lax.GatherDimensionNumbers(
            offset_dims=(), collapsed_slice_dims=(0,), start_index_map=(0,))

        def compute_window(b):
            for j in range(_WIN // 16):
                es = plsc.load_gather(rows_v[b],
                                      [16 * j + i16, jnp.full((16,), 16,
                                                              jnp.int32)])
                ed = plsc.load_gather(dd_v[b],
                                      [16 * j + i16,
                                       jnp.zeros((16,), jnp.int32)])
                e = es + ed
                e = jnp.where(e > 0.0, e, 0.2 * e)
                wj = jnp.exp(e)
                w_sc[b][pl.ds(16 * j, 16)] = wj
                dstsc_v[b][pl.ds(16 * j, 16)] = dst_v[b][pl.ds(16 * j, 16)]
                for l in range(16):
                    k = 16 * j + l
                    bc = lax.gather(
                        wj, jnp.full((16, 1), l, jnp.int32), gdn,
                        slice_sizes=(1,),
                        mode=lax.GatherScatterMode.PROMISE_IN_BOUNDS)
                    sc_rows[b][k, :] = rows_v[b][k, pl.ds(0, 16)] * bc

        def issue_loads(bb, off2):
            pltpu.async_copy(srcp.at[pl.ds(off2, _WIN)], src_v[bb], sem_ld[bb])
            pltpu.async_copy(dstp.at[pl.ds(off2, _WIN)], dst_v[bb], sem_ld[bb])

        def wait_loads(bb, off2):
            mk(srcp.at[pl.ds(off2, _WIN)], src_v[bb], sem_ld[bb]).wait()
            mk(dstp.at[pl.ds(off2, _WIN)], dst_v[bb], sem_ld[bb]).wait()

        def issue_gather(bb):
            for j in range(_WIN // 16):
                srch_v[bb][pl.ds(16 * j, 16)] = (src_v[bb][pl.ds(16 * j, 16)]
                                                 + coff)
            pltpu.async_copy(h2a.at[srch_v[bb]], rows_v[bb], sem_g[bb])
            pltpu.async_copy(adst_tab.at[dst_v[bb]], dd_v[bb], sem_g[bb])

        def wait_gather(b):
            mk(h2a.at[srch_v[b]], rows_v[b], sem_g[b]).wait()
            mk(adst_tab.at[dst_v[b]], dd_v[b], sem_g[b]).wait()

        def issue_scatter(b):
            pltpu.async_copy(sc_rows[b], acc_spm.at[dstsc_v[b]], sem_sc[b],
                             add=True)
            pltpu.async_copy(w_sc[b], den_spm.at[dstsc_v[b]], sem_sc[b],
                             add=True)

        def wait_scatter(b):
            mk(sc_rows[b], acc_spm.at[dstsc_v[b]], sem_sc[b]).wait()
            mk(w_sc[b], den_spm.at[dstsc_v[b]], sem_sc[b]).wait()

        for b in range(3):
            for k in range(_WIN):
                sc_rows[b][k, :] = z16
            for j in range(_WIN // 16):
                w_sc[b][pl.ds(16 * j, 16)] = z16
                dstsc_v[b][pl.ds(16 * j, 16)] = (
                    jnp.full((16,), np_rows - 16, jnp.int32) + i16)
        rbase = sid * rows_per_tile

        def zero_body(i, _):
            pltpu.sync_copy(sc_rows[0],
                            acc_spm.at[pl.ds(rbase + i * _WIN, _WIN)])
            pltpu.sync_copy(w_sc[0], den_spm.at[pl.ds(rbase + i * _WIN, _WIN)])
            return 0

        lax.fori_loop(0, full_chunks, zero_body, 0)
        if tail:
            pltpu.sync_copy(sc_rows[0].at[pl.ds(0, tail)],
                            acc_spm.at[pl.ds(rbase + full_chunks * _WIN, tail)])
            pltpu.sync_copy(w_sc[0].at[pl.ds(0, tail)],
                            den_spm.at[pl.ds(rbase + full_chunks * _WIN, tail)])
        plsc.subcore_barrier()

        ebase = sid * (wins_per_tile * _WIN)
        for b in range(3):
            issue_scatter(b)
        pltpu.sync_copy(srcp.at[pl.ds(ebase, _WIN)], src_v[0])
        pltpu.sync_copy(dstp.at[pl.ds(ebase, _WIN)], dst_v[0])
        issue_loads(1, ebase + _WIN)
        issue_gather(0)

        def win_body(i, _):
            for b in range(3):
                g = 3 * i + b
                b2 = (b + 1) % 3
                b3 = (b + 2) % 3
                wait_gather(b)
                wait_scatter(b)
                compute_window(b)
                issue_scatter(b)
                wait_loads(b2, ebase + (g + 1) * _WIN)
                issue_gather(b2)
                issue_loads(b3, ebase + (g + 2) * _WIN)
            return 0

        lax.fori_loop(0, wins_per_tile // 3, win_body, 0)
        wg = wins_per_tile % 3
        mk(h2a.at[srch_v[wg]], rows_v[wg], sem_g[wg]).wait()
        mk(adst_tab.at[dst_v[wg]], dd_v[wg], sem_g[wg]).wait()
        wl = (wins_per_tile + 1) % 3
        wait_loads(wl, ebase + (wins_per_tile + 1) * _WIN)
        for b in range(3):
            wait_scatter(b)
        plsc.subcore_barrier()

        def epi_body(i, _):
            r0 = sid * rows_per_tile + i * _WIN
            o0 = coff + r0
            pltpu.sync_copy(acc_spm.at[pl.ds(r0, _WIN)], dd_v[0])
            pltpu.sync_copy(dd_v[0], out2.at[pl.ds(o0, _WIN)])
            pltpu.sync_copy(den_spm.at[pl.ds(r0, _WIN)], w_sc[0])
            pltpu.sync_copy(w_sc[0], den2.at[pl.ds(o0, _WIN)])
            return 0

        lax.fori_loop(0, full_chunks, epi_body, 0)
        if tail:
            r0 = sid * rows_per_tile + full_chunks * _WIN
            o0 = coff + r0
            pltpu.sync_copy(acc_spm.at[pl.ds(r0, tail)],
                            dd_v[0].at[pl.ds(0, tail)])
            pltpu.sync_copy(dd_v[0].at[pl.ds(0, tail)],
                            out2.at[pl.ds(o0, tail)])
            pltpu.sync_copy(den_spm.at[pl.ds(r0, tail)],
                            w_sc[0].at[pl.ds(0, tail)])
            pltpu.sync_copy(w_sc[0].at[pl.ds(0, tail)],
                            den2.at[pl.ds(o0, tail)])

    return acc_kernel


def kernel(s, a, edge_index, W_gat, att_src, att_dst, b_gat,
           W1, b1, W2, b2, W3, b3, V1, c1, V2, c2, V3, c3):
    n, obs = s.shape
    act = a.shape[1]
    hdim = W_gat.shape[1]
    Ws = W_gat[:obs]
    Wa = W_gat[obs:]
    B = 2000
    grid = (n // B,)

    full = lambda shp: pl.BlockSpec(shp, lambda i: tuple(0 for _ in shp))
    hlo, hhi, asrc, adst = pl.pallas_call(
        _pre_body,
        grid=grid,
        in_specs=[
            pl.BlockSpec((B, obs), lambda i: (i, 0)),
            pl.BlockSpec((B, act), lambda i: (i, 0)),
            full((obs, hdim)), full((act, hdim)),
            full((1, hdim)), full((1, hdim)),
        ],
        out_specs=[
            pl.BlockSpec((B, 16), lambda i: (i, 0)),
            pl.BlockSpec((B, 16), lambda i: (i, 0)),
            pl.BlockSpec((B, 1), lambda i: (i, 0)),
            pl.BlockSpec((B, 1), lambda i: (i, 0)),
        ],
        out_shape=[
            jax.ShapeDtypeStruct((n, 16), jnp.float32),
            jax.ShapeDtypeStruct((n, 16), jnp.float32),
            jax.ShapeDtypeStruct((n, 1), jnp.float32),
            jax.ShapeDtypeStruct((n, 1), jnp.float32),
        ],
    )(s, a, Ws, Wa, att_src.reshape(1, hdim), att_dst.reshape(1, hdim))

    et = edge_index.shape[1] + n
    wins_per_tile = -(-et // (_NTILE * _WIN * 3)) * 3
    et_pad = wins_per_tile * _NTILE * _WIN
    et_alloc = et_pad + 2 * _NTILE * _WIN
    npad = et_alloc - et
    np_rows = -(-(n + 16) // (_NTILE * 8)) * (_NTILE * 8)

    loops = jnp.arange(n, dtype=jnp.int32)
    srcp = jnp.concatenate([edge_index[0], loops,
                            jnp.zeros((npad,), jnp.int32)])
    dstp = jnp.concatenate([edge_index[1], loops,
                            n + (jnp.arange(npad, dtype=jnp.int32) % 16)])
    asrc_p = jnp.pad(asrc.reshape(n), (0, np_rows - n))
    adst_p = jnp.pad(adst.reshape(n), (0, np_rows - n))
    pad16 = ((0, np_rows - n), (0, 0))
    h2 = jnp.concatenate([jnp.pad(hlo, pad16), jnp.pad(hhi, pad16)])
    asrc2 = jnp.concatenate([asrc_p, asrc_p]).reshape(2 * np_rows, 1)
    h2a = jnp.concatenate(
        [h2, asrc2, jnp.zeros((2 * np_rows, 15), jnp.float32)], axis=1)
    adst_tab = jnp.concatenate(
        [adst_p.reshape(np_rows, 1), jnp.zeros((np_rows, 15), jnp.float32)],
        axis=1)

    acc_kernel = _make_acc_kernel(np_rows, wins_per_tile)
    out2, den2 = acc_kernel(srcp, dstp, adst_tab, h2a)

    lo = out2[:n]
    hi = out2[np_rows:np_rows + n]
    den = den2[:n].reshape(n, 1)

    q1, q2 = pl.pallas_call(
        _post_body,
        grid=grid,
        in_specs=[
            pl.BlockSpec((B, 16), lambda i: (i, 0)),
            pl.BlockSpec((B, 16), lambda i: (i, 0)),
            pl.BlockSpec((B, 1), lambda i: (i, 0)),
            full((1, hdim)),
            full((hdim, hdim)), full((1, hdim)),
            full((hdim, hdim)), full((1, hdim)),
            full((hdim, 1)), full((1, 1)),
            full((hdim, hdim)), full((1, hdim)),
            full((hdim, hdim)), full((1, hdim)),
            full((hdim, 1)), full((1, 1)),
        ],
        out_specs=[
            pl.BlockSpec((B, 1), lambda i: (i, 0)),
            pl.BlockSpec((B, 1), lambda i: (i, 0)),
        ],
        out_shape=[
            jax.ShapeDtypeStruct((n, 1), jnp.float32),
            jax.ShapeDtypeStruct((n, 1), jnp.float32),
        ],
    )(lo, hi, den, b_gat.reshape(1, hdim),
      W1, b1.reshape(1, hdim), W2, b2.reshape(1, hdim), W3, b3.reshape(1, 1),
      V1, c1.reshape(1, hdim), V2, c2.reshape(1, hdim), V3, c3.reshape(1, 1))
    return (q1, q2)
